# baseline (speedup 1.0000x reference)
import functools

import jax
import jax.numpy as jnp
from jax import lax
from jax.experimental import pallas as pl
from jax.experimental.pallas import tpu as pltpu
from jax.experimental.pallas import tpu_sc as plsc

_NC = 80
_BALANCE = (0.5, 1.0, 2.0)
_ALPHA = 0.25
_GAMMA = 2.0
_S = 3
_N = 33600
_T = 256
_RB = 1600
_NB = _N // _RB


def _xyxy(cx, cy, w, h):
    return cx - w * 0.5, cy - h * 0.5, cx + w * 0.5, cy + h * 0.5


def _pass1_body(p_ref, tt_ref, best_ref, bce_ref, rmax, ridx, cbuf):
    b = pl.program_id(1)

    tx1, ty1, tx2, ty2 = _xyxy(tt_ref[1:2, :], tt_ref[2:3, :],
                               tt_ref[3:4, :], tt_ref[4:5, :])
    ta = (tx2 - tx1) * (ty2 - ty1)

    px1, py1, px2, py2 = _xyxy(p_ref[0, :, 80:81], p_ref[0, :, 81:82],
                               p_ref[0, :, 82:83], p_ref[0, :, 83:84])
    pa = (px2 - px1) * (py2 - py1)

    ix1 = jnp.maximum(px1, tx1)
    iy1 = jnp.maximum(py1, ty1)
    ix2 = jnp.minimum(px2, tx2)
    iy2 = jnp.minimum(py2, ty2)
    inter = jnp.maximum(ix2 - ix1, 0.0) * jnp.maximum(iy2 - iy1, 0.0)
    iou = inter / (pa + ta - inter + 1e-7)

    bm = jnp.max(iou, axis=0, keepdims=True)
    rio = lax.broadcasted_iota(jnp.int32, (_RB, _T), 0)
    li = jnp.min(jnp.where(iou == bm, rio, _RB), axis=0, keepdims=True)
    cand = li + b * _RB

    @pl.when(b == 0)
    def _():
        rmax[...] = jnp.full((1, _T), -1.0, jnp.float32)
        ridx[...] = jnp.zeros((1, _T), jnp.int32)
        cbuf[...] = jnp.full((_RB, 32), -1.0, jnp.float32)

    better = bm > rmax[...]
    ridx[...] = jnp.where(better, cand, ridx[...])
    rmax[...] = jnp.where(better, bm, rmax[...])

    lane = lax.broadcasted_iota(jnp.int32, (_RB, 32), 1)
    cbuf[...] = jnp.where(lane == b, p_ref[0, :, 84:85], cbuf[...])

    @pl.when(b == _NB - 1)
    def _():
        xs = cbuf[...]
        sp = jnp.maximum(xs, 0.0) + jnp.log1p(jnp.exp(-jnp.abs(xs)))
        sp = jnp.where(lane < _NB, sp, 0.0)
        best_ref[0, :, :] = ridx[...]
        bce_ref[0, :, :] = jnp.full((1, 1), 0.0, jnp.float32) + jnp.sum(sp)


def _pass1(preds_flat, flat_tt):
    return pl.pallas_call(
        _pass1_body,
        grid=(_S, _NB),
        in_specs=[
            pl.BlockSpec((1, _RB, 85), lambda s, b: (s, b, 0)),
            pl.BlockSpec((6, _T), lambda s, b: (0, 0)),
        ],
        out_specs=[
            pl.BlockSpec((1, 1, _T), lambda s, b: (s, 0, 0)),
            pl.BlockSpec((1, 1, 1), lambda s, b: (s, 0, 0)),
        ],
        out_shape=[
            jax.ShapeDtypeStruct((_S, 1, _T), jnp.int32),
            jax.ShapeDtypeStruct((_S, 1, 1), jnp.float32),
        ],
        scratch_shapes=[
            pltpu.VMEM((1, _T), jnp.float32),
            pltpu.VMEM((1, _T), jnp.int32),
            pltpu.VMEM((_RB, 32), jnp.float32),
        ],
    )(preds_flat, flat_tt)


def _repack_body(in_ref, out_ref):
    out_ref[:, 0:85] = in_ref[...]


def _repack(preds2d):
    blk = 1600
    return pl.pallas_call(
        _repack_body,
        grid=(preds2d.shape[0] // blk,),
        in_specs=[pl.BlockSpec((blk, 85), lambda i: (i, 0))],
        out_specs=pl.BlockSpec((blk, 128), lambda i: (i, 0)),
        out_shape=jax.ShapeDtypeStruct((preds2d.shape[0], 128), jnp.float32),
    )(preds2d)


def _sc_gather(table, gidx):
    info = plsc.get_sparse_core_info()
    nw = info.num_cores * info.num_subcores
    b_total = gidx.shape[0]
    bpw = b_total // nw
    d = table.shape[1]

    mesh = plsc.VectorSubcoreMesh(core_axis_name="c", subcore_axis_name="s")

    @functools.partial(
        pl.kernel,
        mesh=mesh,
        out_type=jax.ShapeDtypeStruct((b_total, d), jnp.float32),
        scratch_types=[
            pltpu.VMEM((bpw,), jnp.int32),
            pltpu.VMEM((bpw, d), jnp.float32),
            pltpu.SemaphoreType.DMA,
        ],
    )
    def gather(table_hbm, idx_hbm, out_hbm, idx_v, rows_v, sem):
        wid = lax.axis_index("s") * info.num_cores + lax.axis_index("c")
        base = wid * bpw
        pltpu.sync_copy(idx_hbm.at[pl.ds(base, bpw)], idx_v)
        pltpu.async_copy(table_hbm.at[idx_v], rows_v, sem).wait()
        pltpu.sync_copy(rows_v, out_hbm.at[pl.ds(base, bpw)])

    return gather(table, gidx)


def _final_body(g_ref, t_ref, tt_ref, b_ref, bt_ref, bce_ref, out_ref):
    m = (t_ref[:, 5:6] > 0.0).astype(jnp.float32)
    mrow = (tt_ref[5:6, :] > 0.0).astype(jnp.float32)
    count = jnp.sum(m)
    denom = jnp.maximum(count, 1.0)

    tx1c, ty1c, tx2c, ty2c = _xyxy(t_ref[:, 1:2], t_ref[:, 2:3],
                                   t_ref[:, 3:4], t_ref[:, 4:5])
    tx1r, ty1r, tx2r, ty2r = _xyxy(tt_ref[1:2, :], tt_ref[2:3, :],
                                   tt_ref[3:4, :], tt_ref[4:5, :])
    ta_r = (tx2r - tx1r) * (ty2r - ty1r)

    tcls = t_ref[:, 0:1].astype(jnp.int32)
    cio = lax.broadcasted_iota(jnp.int32, (_T, _NC), 1)
    oh = cio == tcls
    tb = t_ref[:, 1:5]
    jidx = lax.broadcasted_iota(jnp.int32, (_T, _T), 0)
    kidx = lax.broadcasted_iota(jnp.int32, (_T, _T), 1)
    prior = kidx < jidx

    total = jnp.float32(0.0)
    for s in range(_S):
        gs = g_ref[s]
        vcls = gs[:, 0:_NC]
        vb = gs[:, _NC:_NC + 4]
        vconf = gs[:, 84:85]

        pt = jnp.where(oh, vcls, 1.0 - vcls)
        fl = -_ALPHA * (1.0 - pt) * (1.0 - pt) * jnp.log(pt + 1e-7)
        cls_loss = jnp.sum(fl * m) / (denom * _NC)

        px1, py1, px2, py2 = _xyxy(gs[:, 80:81], gs[:, 81:82],
                                   gs[:, 82:83], gs[:, 83:84])
        pa = (px2 - px1) * (py2 - py1)
        ix1 = jnp.maximum(px1, tx1r)
        iy1 = jnp.maximum(py1, ty1r)
        ix2 = jnp.minimum(px2, tx2r)
        iy2 = jnp.minimum(py2, ty2r)
        inter = jnp.maximum(ix2 - ix1, 0.0) * jnp.maximum(iy2 - iy1, 0.0)
        pair_iou = inter / (pa + ta_r - inter + 1e-7)
        mean_iou = jnp.sum(pair_iou * (m * mrow)) / (denom * denom)

        dlt = jnp.abs(vb - tb)
        l1 = jnp.where(dlt < 1.0, 0.5 * dlt * dlt, dlt - 0.5)
        sl1 = jnp.sum(l1 * m) / (denom * 4.0)
        bbox_loss = (1.0 - mean_iou) + sl1

        bcol = bt_ref[:, s:s + 1]
        brow = b_ref[s:s + 1, :]
        same = (bcol == brow).astype(jnp.float32)
        dup = jnp.sum(same * prior.astype(jnp.float32) * mrow,
                      axis=1, keepdims=True)
        w = m * (dup == 0.0).astype(jnp.float32)
        dsum = jnp.sum(w * vconf)
        bce_s = jnp.sum(bce_ref[s:s + 1, 0:1])
        conf_loss = (bce_s - dsum) / jnp.float32(_N)

        cls_loss = jnp.where(count > 0, cls_loss, 0.0)
        bbox_loss = jnp.where(count > 0, bbox_loss, 0.0)
        total = total + (cls_loss + bbox_loss + conf_loss) * _BALANCE[s]

    out_ref[...] = jnp.full((1, 1), 0.0, jnp.float32) + total / jnp.float32(_S)


def _final(g, flat_t, flat_tt, best, best_t, bce):
    return pl.pallas_call(
        _final_body,
        out_shape=jax.ShapeDtypeStruct((1, 1), jnp.float32),
    )(g, flat_t, flat_tt, best, best_t, bce)


def kernel(predictions, targets):
    preds_flat = predictions.reshape(_S, _N, 85)
    flat_t = targets.reshape(_T, 6)
    flat_tt = flat_t.T

    best3, bce3 = _pass1(preds_flat, flat_tt)
    best = best3.reshape(_S, _T)
    bce = bce3.reshape(_S, 1)

    gidx = (best + jnp.arange(_S, dtype=jnp.int32)[:, None] * _N).reshape(-1)
    table = _repack(preds_flat.reshape(_S * _N, 85))
    g = _sc_gather(table, gidx).reshape(_S, _T, 128)

    out = _final(g, flat_t, flat_tt, best, best.T, bce)
    return out[0, 0]


# speedup vs baseline: 1.8182x; 1.8182x over previous
import jax
import jax.numpy as jnp
from jax import lax
from jax.experimental import pallas as pl
from jax.experimental.pallas import tpu as pltpu

_NC = 80
_BALANCE = (0.5, 1.0, 2.0)
_ALPHA = 0.25
_S = 3
_N = 33600
_T = 256
_R = 1024
_P = 33792
_NB = _P // _R
_R2 = 4224
_NB2 = _P // _R2


def _xyxy(cx, cy, w, h):
    return cx - w * 0.5, cy - h * 0.5, cx + w * 0.5, cy + h * 0.5


def _pass1_body(p_ref, t_ref, best_ref, bce_ref, rmax, ridx, acc):
    b = pl.program_id(1)

    tx1, ty1, tx2, ty2 = _xyxy(t_ref[:, 1:2], t_ref[:, 2:3],
                               t_ref[:, 3:4], t_ref[:, 4:5])
    ta = (tx2 - tx1) * (ty2 - ty1)

    px1, py1, px2, py2 = _xyxy(p_ref[0, 0:1, :], p_ref[0, 1:2, :],
                               p_ref[0, 2:3, :], p_ref[0, 3:4, :])
    pa = (px2 - px1) * (py2 - py1)

    ix1 = jnp.maximum(px1, tx1)
    iy1 = jnp.maximum(py1, ty1)
    ix2 = jnp.minimum(px2, tx2)
    iy2 = jnp.minimum(py2, ty2)
    inter = jnp.maximum(ix2 - ix1, 0.0) * jnp.maximum(iy2 - iy1, 0.0)
    iou = inter / (pa + ta - inter + 1e-7)

    bm = jnp.max(iou, axis=1, keepdims=True)
    lane = lax.broadcasted_iota(jnp.int32, (_T, _R), 1)
    li = jnp.min(jnp.where(iou == bm, lane, _R), axis=1, keepdims=True)
    cand = li + b * _R

    @pl.when(b == 0)
    def _():
        rmax[...] = jnp.full((_T, 1), -1.0, jnp.float32)
        ridx[...] = jnp.zeros((_T, 1), jnp.int32)
        acc[...] = jnp.zeros((1, 1), jnp.float32)

    better = bm > rmax[...]
    ridx[...] = jnp.where(better, cand, ridx[...])
    rmax[...] = jnp.where(better, bm, rmax[...])

    x = p_ref[0, 4:5, :]
    acc[...] += jnp.sum(jnp.maximum(x, 0.0) + jnp.log1p(jnp.exp(-jnp.abs(x))))

    @pl.when(b == _NB - 1)
    def _():
        best_ref[0, :, :] = ridx[...]
        bce_ref[0, :, :] = acc[...]


def _pass1(ptp, flat_t):
    return pl.pallas_call(
        _pass1_body,
        grid=(_S, _NB),
        in_specs=[
            pl.BlockSpec((1, 8, _R), lambda s, b: (s, 10, b)),
            pl.BlockSpec((_T, 6), lambda s, b: (0, 0)),
        ],
        out_specs=[
            pl.BlockSpec((1, _T, 1), lambda s, b: (s, 0, 0)),
            pl.BlockSpec((1, 1, 1), lambda s, b: (s, 0, 0)),
        ],
        out_shape=[
            jax.ShapeDtypeStruct((_S, _T, 1), jnp.int32),
            jax.ShapeDtypeStruct((_S, 1, 1), jnp.float32),
        ],
        scratch_shapes=[
            pltpu.VMEM((_T, 1), jnp.float32),
            pltpu.VMEM((_T, 1), jnp.int32),
            pltpu.VMEM((1, 1), jnp.float32),
        ],
    )(ptp, flat_t)


def _gather_body(p_ref, best_ref, g_ref):
    b = pl.program_id(1)
    bcol = best_ref[0]
    gio = lax.broadcasted_iota(jnp.int32, (_T, _R2), 1) + b * _R2
    oh = (gio == bcol).astype(jnp.float32)
    part = jax.lax.dot_general(oh, p_ref[0], (((1,), (1,)), ((), ())),
                               preferred_element_type=jnp.float32)

    @pl.when(b == 0)
    def _():
        g_ref[0] = part

    @pl.when(b > 0)
    def _():
        g_ref[0] += part


def _gather(ptp, best3):
    return pl.pallas_call(
        _gather_body,
        grid=(_S, _NB2),
        in_specs=[
            pl.BlockSpec((1, 88, _R2), lambda s, b: (s, 0, b)),
            pl.BlockSpec((1, _T, 1), lambda s, b: (s, 0, 0)),
        ],
        out_specs=pl.BlockSpec((1, _T, 88), lambda s, b: (s, 0, 0)),
        out_shape=jax.ShapeDtypeStruct((_S, _T, 88), jnp.float32),
    )(ptp, best3)


def _final_body(g_ref, t_ref, tt_ref, b_ref, bt_ref, bce_ref, out_ref):
    m = (t_ref[:, 5:6] > 0.0).astype(jnp.float32)
    mrow = (tt_ref[5:6, :] > 0.0).astype(jnp.float32)
    count = jnp.sum(m)
    denom = jnp.maximum(count, 1.0)

    tx1r, ty1r, tx2r, ty2r = _xyxy(tt_ref[1:2, :], tt_ref[2:3, :],
                                   tt_ref[3:4, :], tt_ref[4:5, :])
    ta_r = (tx2r - tx1r) * (ty2r - ty1r)

    tcls = t_ref[:, 0:1].astype(jnp.int32)
    cio = lax.broadcasted_iota(jnp.int32, (_T, _NC), 1)
    oh = cio == tcls
    tb = t_ref[:, 1:5]
    jidx = lax.broadcasted_iota(jnp.int32, (_T, _T), 0)
    kidx = lax.broadcasted_iota(jnp.int32, (_T, _T), 1)
    prior = (kidx < jidx).astype(jnp.float32)

    total = jnp.float32(0.0)
    for s in range(_S):
        gs = g_ref[s]
        vcls = gs[:, 0:_NC]
        vb = gs[:, _NC:_NC + 4]
        vconf = gs[:, 84:85]

        pt = jnp.where(oh, vcls, 1.0 - vcls)
        fl = -_ALPHA * (1.0 - pt) * (1.0 - pt) * jnp.log(pt + 1e-7)
        cls_loss = jnp.sum(fl * m) / (denom * _NC)

        px1, py1, px2, py2 = _xyxy(gs[:, 80:81], gs[:, 81:82],
                                   gs[:, 82:83], gs[:, 83:84])
        pa = (px2 - px1) * (py2 - py1)
        ix1 = jnp.maximum(px1, tx1r)
        iy1 = jnp.maximum(py1, ty1r)
        ix2 = jnp.minimum(px2, tx2r)
        iy2 = jnp.minimum(py2, ty2r)
        inter = jnp.maximum(ix2 - ix1, 0.0) * jnp.maximum(iy2 - iy1, 0.0)
        pair_iou = inter / (pa + ta_r - inter + 1e-7)
        mean_iou = jnp.sum(pair_iou * (m * mrow)) / (denom * denom)

        dlt = jnp.abs(vb - tb)
        l1 = jnp.where(dlt < 1.0, 0.5 * dlt * dlt, dlt - 0.5)
        sl1 = jnp.sum(l1 * m) / (denom * 4.0)
        bbox_loss = (1.0 - mean_iou) + sl1

        bcol = bt_ref[:, s:s + 1]
        brow = b_ref[s:s + 1, :]
        same = (bcol == brow).astype(jnp.float32)
        dup = jnp.sum(same * prior * mrow, axis=1, keepdims=True)
        w = m * (dup == 0.0).astype(jnp.float32)
        dsum = jnp.sum(w * vconf)
        bce_s = jnp.sum(bce_ref[s:s + 1, 0:1])
        conf_loss = (bce_s - dsum) / jnp.float32(_N)

        cls_loss = jnp.where(count > 0, cls_loss, 0.0)
        bbox_loss = jnp.where(count > 0, bbox_loss, 0.0)
        total = total + (cls_loss + bbox_loss + conf_loss) * _BALANCE[s]

    out_ref[...] = jnp.full((1, 1), 0.0, jnp.float32) + total / jnp.float32(_S)


def _final(g, flat_t, flat_tt, best, best_t, bce):
    return pl.pallas_call(
        _final_body,
        out_shape=jax.ShapeDtypeStruct((1, 1), jnp.float32),
    )(g, flat_t, flat_tt, best, best_t, bce)


def kernel(predictions, targets):
    flat_t = targets.reshape(_T, 6)
    flat_tt = flat_t.T

    pt = jnp.transpose(predictions, (0, 3, 1, 2)).reshape(_S, 85, _N)
    ptp = jnp.pad(pt, ((0, 0), (0, 3), (0, _P - _N)),
                  constant_values=-1e30)

    best3, bce3 = _pass1(ptp, flat_t)
    best = best3.reshape(_S, _T)
    bce = bce3.reshape(_S, 1)

    g = _gather(ptp, best3)

    out = _final(g, flat_t, flat_tt, best, best.T, bce)
    return out[0, 0]


# speedup vs baseline: 2.1272x; 1.1700x over previous
import jax
import jax.numpy as jnp
from jax import lax
from jax.experimental import pallas as pl
from jax.experimental.pallas import tpu as pltpu

_NC = 80
_BALANCE = (0.5, 1.0, 2.0)
_ALPHA = 0.25
_S = 3
_N = 33600
_T = 256
_R = 4224
_P = 33792
_NB = _P // _R
_R2 = 4224
_NB2 = _P // _R2


def _xyxy(cx, cy, w, h):
    return cx - w * 0.5, cy - h * 0.5, cx + w * 0.5, cy + h * 0.5


def _pass1_body(p_ref, t_ref, best_ref, bce_ref, rmax, ridx, acc):
    b = pl.program_id(1)

    tx1, ty1, tx2, ty2 = _xyxy(t_ref[:, 1:2], t_ref[:, 2:3],
                               t_ref[:, 3:4], t_ref[:, 4:5])
    ta = (tx2 - tx1) * (ty2 - ty1)

    px1, py1, px2, py2 = _xyxy(p_ref[0, 0:1, :], p_ref[0, 1:2, :],
                               p_ref[0, 2:3, :], p_ref[0, 3:4, :])
    pa = (px2 - px1) * (py2 - py1)

    ix1 = jnp.maximum(px1, tx1)
    iy1 = jnp.maximum(py1, ty1)
    ix2 = jnp.minimum(px2, tx2)
    iy2 = jnp.minimum(py2, ty2)
    inter = jnp.maximum(ix2 - ix1, 0.0) * jnp.maximum(iy2 - iy1, 0.0)
    iou = inter / (pa + ta - inter + 1e-7)

    bm = jnp.max(iou, axis=1, keepdims=True)
    lane = lax.broadcasted_iota(jnp.int32, (_T, _R), 1)
    li = jnp.min(jnp.where(iou == bm, lane, _R), axis=1, keepdims=True)
    cand = li + b * _R

    @pl.when(b == 0)
    def _():
        rmax[...] = jnp.full((_T, 1), -1.0, jnp.float32)
        ridx[...] = jnp.zeros((_T, 1), jnp.int32)
        acc[...] = jnp.zeros((1, 1), jnp.float32)

    better = bm > rmax[...]
    ridx[...] = jnp.where(better, cand, ridx[...])
    rmax[...] = jnp.where(better, bm, rmax[...])

    x = p_ref[0, 4:5, :]
    acc[...] += jnp.sum(jnp.maximum(x, 0.0) + jnp.log1p(jnp.exp(-jnp.abs(x))))

    @pl.when(b == _NB - 1)
    def _():
        best_ref[0, :, :] = ridx[...]
        bce_ref[0, :, :] = acc[...]


def _pass1(ptp, flat_t):
    return pl.pallas_call(
        _pass1_body,
        grid=(_S, _NB),
        in_specs=[
            pl.BlockSpec((1, 8, _R), lambda s, b: (s, 10, b)),
            pl.BlockSpec((_T, 6), lambda s, b: (0, 0)),
        ],
        out_specs=[
            pl.BlockSpec((1, _T, 1), lambda s, b: (s, 0, 0)),
            pl.BlockSpec((1, 1, 1), lambda s, b: (s, 0, 0)),
        ],
        out_shape=[
            jax.ShapeDtypeStruct((_S, _T, 1), jnp.int32),
            jax.ShapeDtypeStruct((_S, 1, 1), jnp.float32),
        ],
        scratch_shapes=[
            pltpu.VMEM((_T, 1), jnp.float32),
            pltpu.VMEM((_T, 1), jnp.int32),
            pltpu.VMEM((1, 1), jnp.float32),
        ],
    )(ptp, flat_t)


def _gather_body(p_ref, best_ref, g_ref):
    b = pl.program_id(1)
    bcol = best_ref[0]
    gio = lax.broadcasted_iota(jnp.int32, (_T, _R2), 1) + b * _R2
    oh = (gio == bcol).astype(jnp.float32)
    part = jax.lax.dot_general(oh, p_ref[0], (((1,), (1,)), ((), ())),
                               preferred_element_type=jnp.float32)

    @pl.when(b == 0)
    def _():
        g_ref[0] = part

    @pl.when(b > 0)
    def _():
        g_ref[0] += part


def _gather(ptp, best3):
    return pl.pallas_call(
        _gather_body,
        grid=(_S, _NB2),
        in_specs=[
            pl.BlockSpec((1, 88, _R2), lambda s, b: (s, 0, b)),
            pl.BlockSpec((1, _T, 1), lambda s, b: (s, 0, 0)),
        ],
        out_specs=pl.BlockSpec((1, _T, 88), lambda s, b: (s, 0, 0)),
        out_shape=jax.ShapeDtypeStruct((_S, _T, 88), jnp.float32),
    )(ptp, best3)


def _final_body(g_ref, t_ref, tt_ref, b_ref, bt_ref, bce_ref, out_ref):
    m = (t_ref[:, 5:6] > 0.0).astype(jnp.float32)
    mrow = (tt_ref[5:6, :] > 0.0).astype(jnp.float32)
    count = jnp.sum(m)
    denom = jnp.maximum(count, 1.0)

    tx1r, ty1r, tx2r, ty2r = _xyxy(tt_ref[1:2, :], tt_ref[2:3, :],
                                   tt_ref[3:4, :], tt_ref[4:5, :])
    ta_r = (tx2r - tx1r) * (ty2r - ty1r)

    tcls = t_ref[:, 0:1].astype(jnp.int32)
    cio = lax.broadcasted_iota(jnp.int32, (_T, _NC), 1)
    oh = cio == tcls
    tb = t_ref[:, 1:5]
    jidx = lax.broadcasted_iota(jnp.int32, (_T, _T), 0)
    kidx = lax.broadcasted_iota(jnp.int32, (_T, _T), 1)
    prior = (kidx < jidx).astype(jnp.float32)

    total = jnp.float32(0.0)
    for s in range(_S):
        gs = g_ref[s]
        vcls = gs[:, 0:_NC]
        vb = gs[:, _NC:_NC + 4]
        vconf = gs[:, 84:85]

        pt = jnp.where(oh, vcls, 1.0 - vcls)
        fl = -_ALPHA * (1.0 - pt) * (1.0 - pt) * jnp.log(pt + 1e-7)
        cls_loss = jnp.sum(fl * m) / (denom * _NC)

        px1, py1, px2, py2 = _xyxy(gs[:, 80:81], gs[:, 81:82],
                                   gs[:, 82:83], gs[:, 83:84])
        pa = (px2 - px1) * (py2 - py1)
        ix1 = jnp.maximum(px1, tx1r)
        iy1 = jnp.maximum(py1, ty1r)
        ix2 = jnp.minimum(px2, tx2r)
        iy2 = jnp.minimum(py2, ty2r)
        inter = jnp.maximum(ix2 - ix1, 0.0) * jnp.maximum(iy2 - iy1, 0.0)
        pair_iou = inter / (pa + ta_r - inter + 1e-7)
        mean_iou = jnp.sum(pair_iou * (m * mrow)) / (denom * denom)

        dlt = jnp.abs(vb - tb)
        l1 = jnp.where(dlt < 1.0, 0.5 * dlt * dlt, dlt - 0.5)
        sl1 = jnp.sum(l1 * m) / (denom * 4.0)
        bbox_loss = (1.0 - mean_iou) + sl1

        bcol = bt_ref[:, s:s + 1]
        brow = b_ref[s:s + 1, :]
        same = (bcol == brow).astype(jnp.float32)
        dup = jnp.sum(same * prior * mrow, axis=1, keepdims=True)
        w = m * (dup == 0.0).astype(jnp.float32)
        dsum = jnp.sum(w * vconf)
        bce_s = jnp.sum(bce_ref[s:s + 1, 0:1])
        conf_loss = (bce_s - dsum) / jnp.float32(_N)

        cls_loss = jnp.where(count > 0, cls_loss, 0.0)
        bbox_loss = jnp.where(count > 0, bbox_loss, 0.0)
        total = total + (cls_loss + bbox_loss + conf_loss) * _BALANCE[s]

    out_ref[...] = jnp.full((1, 1), 0.0, jnp.float32) + total / jnp.float32(_S)


def _final(g, flat_t, flat_tt, best, best_t, bce):
    return pl.pallas_call(
        _final_body,
        out_shape=jax.ShapeDtypeStruct((1, 1), jnp.float32),
    )(g, flat_t, flat_tt, best, best_t, bce)


def kernel(predictions, targets):
    flat_t = targets.reshape(_T, 6)
    flat_tt = flat_t.T

    pt = jnp.transpose(predictions, (0, 3, 1, 2)).reshape(_S, 85, _N)
    ptp = jnp.pad(pt, ((0, 0), (0, 3), (0, _P - _N)),
                  constant_values=-1e30)

    best3, bce3 = _pass1(ptp, flat_t)
    best = best3.reshape(_S, _T)
    bce = bce3.reshape(_S, 1)

    g = _gather(ptp, best3)

    out = _final(g, flat_t, flat_tt, best, best.T, bce)
    return out[0, 0]


# speedup vs baseline: 2.1787x; 1.0242x over previous
import jax
import jax.numpy as jnp
from jax import lax
from jax.experimental import pallas as pl
from jax.experimental.pallas import tpu as pltpu

_NC = 80
_BALANCE = (0.5, 1.0, 2.0)
_ALPHA = 0.25
_S = 3
_N = 33600
_T = 256
_R = 8448
_P = 33792
_NB = _P // _R
_R2 = 4224
_NB2 = _P // _R2


def _xyxy(cx, cy, w, h):
    return cx - w * 0.5, cy - h * 0.5, cx + w * 0.5, cy + h * 0.5


def _pass1_body(p_ref, t_ref, best_ref, bce_ref, rmax, ridx, acc):
    b = pl.program_id(1)

    tx1, ty1, tx2, ty2 = _xyxy(t_ref[:, 1:2], t_ref[:, 2:3],
                               t_ref[:, 3:4], t_ref[:, 4:5])
    ta = (tx2 - tx1) * (ty2 - ty1)

    px1, py1, px2, py2 = _xyxy(p_ref[0, 0:1, :], p_ref[0, 1:2, :],
                               p_ref[0, 2:3, :], p_ref[0, 3:4, :])
    pa = (px2 - px1) * (py2 - py1)

    ix1 = jnp.maximum(px1, tx1)
    iy1 = jnp.maximum(py1, ty1)
    ix2 = jnp.minimum(px2, tx2)
    iy2 = jnp.minimum(py2, ty2)
    inter = jnp.maximum(ix2 - ix1, 0.0) * jnp.maximum(iy2 - iy1, 0.0)
    iou = inter / (pa + ta - inter + 1e-7)

    bm = jnp.max(iou, axis=1, keepdims=True)
    lane = lax.broadcasted_iota(jnp.int32, (_T, _R), 1)
    li = jnp.min(jnp.where(iou == bm, lane, _R), axis=1, keepdims=True)
    cand = li + b * _R

    @pl.when(b == 0)
    def _():
        rmax[...] = jnp.full((_T, 1), -1.0, jnp.float32)
        ridx[...] = jnp.zeros((_T, 1), jnp.int32)
        acc[...] = jnp.zeros((1, 1), jnp.float32)

    better = bm > rmax[...]
    ridx[...] = jnp.where(better, cand, ridx[...])
    rmax[...] = jnp.where(better, bm, rmax[...])

    x = p_ref[0, 4:5, :]
    acc[...] += jnp.sum(jnp.maximum(x, 0.0) + jnp.log1p(jnp.exp(-jnp.abs(x))))

    @pl.when(b == _NB - 1)
    def _():
        best_ref[0, :, :] = ridx[...]
        bce_ref[0, :, :] = acc[...]


def _pass1(ptp, flat_t):
    return pl.pallas_call(
        _pass1_body,
        grid=(_S, _NB),
        in_specs=[
            pl.BlockSpec((1, 8, _R), lambda s, b: (s, 10, b)),
            pl.BlockSpec((_T, 6), lambda s, b: (0, 0)),
        ],
        out_specs=[
            pl.BlockSpec((1, _T, 1), lambda s, b: (s, 0, 0)),
            pl.BlockSpec((1, 1, 1), lambda s, b: (s, 0, 0)),
        ],
        out_shape=[
            jax.ShapeDtypeStruct((_S, _T, 1), jnp.int32),
            jax.ShapeDtypeStruct((_S, 1, 1), jnp.float32),
        ],
        scratch_shapes=[
            pltpu.VMEM((_T, 1), jnp.float32),
            pltpu.VMEM((_T, 1), jnp.int32),
            pltpu.VMEM((1, 1), jnp.float32),
        ],
    )(ptp, flat_t)


def _gather_body(p_ref, best_ref, g_ref):
    b = pl.program_id(1)
    bcol = best_ref[0]
    gio = lax.broadcasted_iota(jnp.int32, (_T, _R2), 1) + b * _R2
    oh = (gio == bcol).astype(jnp.float32)
    part = jax.lax.dot_general(oh, p_ref[0], (((1,), (1,)), ((), ())),
                               preferred_element_type=jnp.float32)

    @pl.when(b == 0)
    def _():
        g_ref[0] = part

    @pl.when(b > 0)
    def _():
        g_ref[0] += part


def _gather(ptp, best3):
    return pl.pallas_call(
        _gather_body,
        grid=(_S, _NB2),
        in_specs=[
            pl.BlockSpec((1, 88, _R2), lambda s, b: (s, 0, b)),
            pl.BlockSpec((1, _T, 1), lambda s, b: (s, 0, 0)),
        ],
        out_specs=pl.BlockSpec((1, _T, 88), lambda s, b: (s, 0, 0)),
        out_shape=jax.ShapeDtypeStruct((_S, _T, 88), jnp.float32),
    )(ptp, best3)


def _final_body(g_ref, t_ref, tt_ref, b_ref, bt_ref, bce_ref, out_ref):
    m = (t_ref[:, 5:6] > 0.0).astype(jnp.float32)
    mrow = (tt_ref[5:6, :] > 0.0).astype(jnp.float32)
    count = jnp.sum(m)
    denom = jnp.maximum(count, 1.0)

    tx1r, ty1r, tx2r, ty2r = _xyxy(tt_ref[1:2, :], tt_ref[2:3, :],
                                   tt_ref[3:4, :], tt_ref[4:5, :])
    ta_r = (tx2r - tx1r) * (ty2r - ty1r)

    tcls = t_ref[:, 0:1].astype(jnp.int32)
    cio = lax.broadcasted_iota(jnp.int32, (_T, _NC), 1)
    oh = cio == tcls
    tb = t_ref[:, 1:5]
    jidx = lax.broadcasted_iota(jnp.int32, (_T, _T), 0)
    kidx = lax.broadcasted_iota(jnp.int32, (_T, _T), 1)
    prior = (kidx < jidx).astype(jnp.float32)

    total = jnp.float32(0.0)
    for s in range(_S):
        gs = g_ref[s]
        vcls = gs[:, 0:_NC]
        vb = gs[:, _NC:_NC + 4]
        vconf = gs[:, 84:85]

        pt = jnp.where(oh, vcls, 1.0 - vcls)
        fl = -_ALPHA * (1.0 - pt) * (1.0 - pt) * jnp.log(pt + 1e-7)
        cls_loss = jnp.sum(fl * m) / (denom * _NC)

        px1, py1, px2, py2 = _xyxy(gs[:, 80:81], gs[:, 81:82],
                                   gs[:, 82:83], gs[:, 83:84])
        pa = (px2 - px1) * (py2 - py1)
        ix1 = jnp.maximum(px1, tx1r)
        iy1 = jnp.maximum(py1, ty1r)
        ix2 = jnp.minimum(px2, tx2r)
        iy2 = jnp.minimum(py2, ty2r)
        inter = jnp.maximum(ix2 - ix1, 0.0) * jnp.maximum(iy2 - iy1, 0.0)
        pair_iou = inter / (pa + ta_r - inter + 1e-7)
        mean_iou = jnp.sum(pair_iou * (m * mrow)) / (denom * denom)

        dlt = jnp.abs(vb - tb)
        l1 = jnp.where(dlt < 1.0, 0.5 * dlt * dlt, dlt - 0.5)
        sl1 = jnp.sum(l1 * m) / (denom * 4.0)
        bbox_loss = (1.0 - mean_iou) + sl1

        bcol = bt_ref[:, s:s + 1]
        brow = b_ref[s:s + 1, :]
        same = (bcol == brow).astype(jnp.float32)
        dup = jnp.sum(same * prior * mrow, axis=1, keepdims=True)
        w = m * (dup == 0.0).astype(jnp.float32)
        dsum = jnp.sum(w * vconf)
        bce_s = jnp.sum(bce_ref[s:s + 1, 0:1])
        conf_loss = (bce_s - dsum) / jnp.float32(_N)

        cls_loss = jnp.where(count > 0, cls_loss, 0.0)
        bbox_loss = jnp.where(count > 0, bbox_loss, 0.0)
        total = total + (cls_loss + bbox_loss + conf_loss) * _BALANCE[s]

    out_ref[...] = jnp.full((1, 1), 0.0, jnp.float32) + total / jnp.float32(_S)


def _final(g, flat_t, flat_tt, best, best_t, bce):
    return pl.pallas_call(
        _final_body,
        out_shape=jax.ShapeDtypeStruct((1, 1), jnp.float32),
    )(g, flat_t, flat_tt, best, best_t, bce)


def kernel(predictions, targets):
    flat_t = targets.reshape(_T, 6)
    flat_tt = flat_t.T

    pt = jnp.transpose(predictions, (0, 3, 1, 2)).reshape(_S, 85, _N)
    ptp = jnp.pad(pt, ((0, 0), (0, 3), (0, _P - _N)),
                  constant_values=-1e30)

    best3, bce3 = _pass1(ptp, flat_t)
    best = best3.reshape(_S, _T)
    bce = bce3.reshape(_S, 1)

    g = _gather(ptp, best3)

    out = _final(g, flat_t, flat_tt, best, best.T, bce)
    return out[0, 0]
